# baseline (speedup 1.0000x reference)
import functools

import jax
import jax.numpy as jnp
from jax import lax
from jax.experimental import pallas as pl
from jax.experimental.pallas import tpu as pltpu
from jax.experimental.pallas import tpu_sc as plsc

VOCAB = 1000000
DIM = 64
BATCH = 16384
HIST = 50

_B = BATCH * HIST
_R_ROWS = VOCAB // 2

_info = plsc.get_sparse_core_info()
_NC, _NS = _info.num_cores, _info.num_subcores
_NW = _NC * _NS
_B_PER_W = _B // _NW
_CHUNK = 800
_N_CHUNK = _B_PER_W // _CHUNK
_N_OUTER = _N_CHUNK // 2

_mesh = plsc.VectorSubcoreMesh(core_axis_name="c", subcore_axis_name="s")


_K1_RW = 256
_K1_GRID = -(-_R_ROWS // _K1_RW)
_R_PAD = _K1_GRID * _K1_RW


def _eye64():
    return (lax.broadcasted_iota(jnp.int32, (DIM, DIM), 0)
            == lax.broadcasted_iota(jnp.int32, (DIM, DIM), 1)).astype(jnp.float32)


def _k1_body(a_ref, b_ref, r_ref):
    eye = _eye64()
    dn = (((0,), (0,)), ((), ()))
    r_ref[:, 0:DIM] = lax.dot_general(
        a_ref[...], eye, dn, preferred_element_type=jnp.float32)
    r_ref[:, DIM:128] = lax.dot_general(
        b_ref[...], eye, dn, preferred_element_type=jnp.float32)


_k1_call = pl.pallas_call(
    _k1_body,
    grid=(_K1_GRID,),
    in_specs=[
        pl.BlockSpec((DIM, _K1_RW), lambda j: (0, 2 * j)),
        pl.BlockSpec((DIM, _K1_RW),
                     lambda j: (0, jnp.minimum(2 * j + 1, VOCAB // _K1_RW - 1))),
    ],
    out_specs=pl.BlockSpec((_K1_RW, 128), lambda j: (j, 0)),
    out_shape=jax.ShapeDtypeStruct((_R_PAD, 128), jnp.float32),
)


@functools.partial(
    pl.kernel,
    mesh=_mesh,
    out_type=jax.ShapeDtypeStruct((_B, DIM), jnp.float32),
    scratch_types=[
        pltpu.VMEM((_B_PER_W,), jnp.int32),
        pltpu.VMEM((_CHUNK, DIM), jnp.float32),
        pltpu.VMEM((_CHUNK, DIM), jnp.float32),
        pltpu.SemaphoreType.DMA,
        pltpu.SemaphoreType.DMA,
    ],
    compiler_params=pltpu.CompilerParams(use_tc_tiling_on_sc=False),
)
def _gather_kernel(table_hbm, idx_hbm, out_hbm, idx_v, rows0, rows1, sem0, sem1):
    wid = lax.axis_index("s") * _NC + lax.axis_index("c")
    base = wid * _B_PER_W

    pltpu.sync_copy(idx_hbm.at[pl.ds(base, _B_PER_W)], idx_v)

    def vmap_idx(i, _):
        for j in range(8):
            off = i * 128 + j * 16
            v16 = idx_v[pl.ds(off, 16)]
            idx_v[pl.ds(off, 16)] = (
                lax.shift_left(lax.shift_right_logical(v16, 9), 9)
                + lax.shift_left(jnp.bitwise_and(v16, 255), 1)
                + jnp.bitwise_and(lax.shift_right_logical(v16, 8), 1))
        return _

    lax.fori_loop(0, _B_PER_W // 128, vmap_idx, None)

    def start_gather(s, rows, sem):
        pltpu.async_copy(table_hbm.at[idx_v.at[pl.ds(s * _CHUNK, _CHUNK)]],
                         rows, sem)

    def finish(s, rows, sem):
        pltpu.make_async_copy(
            table_hbm.at[idx_v.at[pl.ds(s * _CHUNK, _CHUNK)]], rows, sem
        ).wait()
        pltpu.sync_copy(rows, out_hbm.at[pl.ds(base + s * _CHUNK, _CHUNK)])

    start_gather(0, rows0, sem0)

    def outer(o, _):
        s0 = 2 * o
        start_gather(s0 + 1, rows1, sem1)
        finish(s0, rows0, sem0)
        start_gather(s0 + 2, rows0, sem0)
        finish(s0 + 1, rows1, sem1)
        return _

    lax.fori_loop(0, _N_OUTER - 1, outer, None)

    s0 = _N_CHUNK - 2
    start_gather(s0 + 1, rows1, sem1)
    finish(s0, rows0, sem0)
    finish(s0 + 1, rows1, sem1)


_K3_BW = 2048
_K3_NB = BATCH // _K3_BW


def _k3_body(g_ref, o_ref):
    eye = _eye64()
    dn = (((0,), (1,)), ((), ()))
    o_ref[0, :, 0:_K3_BW // 2] = lax.dot_general(
        eye, g_ref[:, 0:DIM], dn, preferred_element_type=jnp.float32)
    o_ref[0, :, _K3_BW // 2:_K3_BW] = lax.dot_general(
        eye, g_ref[:, DIM:128], dn, preferred_element_type=jnp.float32)


_k3_call = pl.pallas_call(
    _k3_body,
    grid=(HIST, _K3_NB),
    in_specs=[pl.BlockSpec((_K3_BW // 2, 128),
                           lambda h, jb: (h * _K3_NB + jb, 0))],
    out_specs=pl.BlockSpec((1, DIM, _K3_BW), lambda h, jb: (h, 0, jb)),
    out_shape=jax.ShapeDtypeStruct((HIST, DIM, BATCH), jnp.float32),
)


def kernel(x, embedding):
    r = _k1_call(embedding.T, embedding.T)
    r2d = r.reshape(_R_PAD * 2, DIM)
    xperm = x.T.reshape(HIST, _K3_NB, 2, _K3_BW // 2)
    xperm = xperm.transpose(0, 1, 3, 2).reshape(_B)
    g = _gather_kernel(r2d, xperm)
    g2 = g.reshape(_B // 2, 128)
    outT = _k3_call(g2)
    return outT.transpose(2, 0, 1)


# speedup vs baseline: 1.0069x; 1.0069x over previous
import functools

import jax
import jax.numpy as jnp
from jax import lax
from jax.experimental import pallas as pl
from jax.experimental.pallas import tpu as pltpu
from jax.experimental.pallas import tpu_sc as plsc

VOCAB = 1000000
DIM = 64
BATCH = 16384
HIST = 50

_B = BATCH * HIST
_R_ROWS = VOCAB // 2

_info = plsc.get_sparse_core_info()
_NC, _NS = _info.num_cores, _info.num_subcores
_NW = _NC * _NS

_N_FULL = VOCAB // 128
_TAIL = VOCAB - _N_FULL * 128
_K1_MAIN = 244

_B_PER_W = _B // _NW
_CHUNK = 800
_N_CHUNK = _B_PER_W // _CHUNK
_N_OUTER = _N_CHUNK // 2

_mesh = plsc.VectorSubcoreMesh(core_axis_name="c", subcore_axis_name="s")


def _transpose_chunk(tbuf, rbuf):
    iota16 = lax.iota(jnp.int32, 16)

    def row_step(rr, _):
        for q in (0, 1):
            col = jnp.full((16,), 2 * rr + q, jnp.int32)
            for g in range(4):
                row = iota16 + (16 * g)
                v = plsc.load_gather(tbuf, [row, col])
                rbuf[rr, pl.ds(64 * q + 16 * g, 16)] = v
        return _

    lax.fori_loop(0, 64, row_step, None)


@functools.partial(
    pl.kernel,
    mesh=_mesh,
    out_type=jax.ShapeDtypeStruct((_R_ROWS, 128), jnp.float32),
    scratch_types=[
        pltpu.VMEM((64, 128), jnp.float32),
        pltpu.VMEM((64, 128), jnp.float32),
        pltpu.VMEM((64, 128), jnp.float32),
        pltpu.VMEM((64, 128), jnp.float32),
        pltpu.SemaphoreType.DMA,
        pltpu.SemaphoreType.DMA,
        pltpu.SemaphoreType.DMA,
        pltpu.SemaphoreType.DMA,
    ],
    compiler_params=pltpu.CompilerParams(use_tc_tiling_on_sc=True,
                                         needs_layout_passes=False),
)
def _rebuild_kernel(tableT_hbm, tail_hbm, r_hbm,
                    tb0, tb1, rb0, rb1, sl0, sl1, sw0, sw1):
    wid = lax.axis_index("s") * _NC + lax.axis_index("c")

    def src(k):
        return tableT_hbm.at[:, pl.ds((wid + _NW * k) * 128, 128)]

    def dst(k):
        return r_hbm.at[pl.ds((wid + _NW * k) * 64, 64), :]

    def load(k, tb, sl):
        pltpu.async_copy(src(k), tb, sl)

    def wload(k, tb, sl):
        pltpu.make_async_copy(src(k), tb, sl).wait()

    def write(k, rb, sw):
        pltpu.async_copy(rb, dst(k), sw)

    def wwrite(k, rb, sw):
        pltpu.make_async_copy(rb, dst(k), sw).wait()

    load(0, tb0, sl0)
    load(1, tb1, sl1)

    wload(0, tb0, sl0)
    _transpose_chunk(tb0, rb0)
    write(0, rb0, sw0)
    load(2, tb0, sl0)
    wload(1, tb1, sl1)
    _transpose_chunk(tb1, rb1)
    write(1, rb1, sw1)
    load(3, tb1, sl1)

    def outer(p, _):
        k0 = 2 * p
        wload(k0, tb0, sl0)
        wwrite(k0 - 2, rb0, sw0)
        _transpose_chunk(tb0, rb0)
        write(k0, rb0, sw0)
        load(k0 + 2, tb0, sl0)
        wload(k0 + 1, tb1, sl1)
        wwrite(k0 - 1, rb1, sw1)
        _transpose_chunk(tb1, rb1)
        write(k0 + 1, rb1, sw1)
        load(k0 + 3, tb1, sl1)
        return _

    lax.fori_loop(1, _K1_MAIN // 2 - 1, outer, None)

    k0 = _K1_MAIN - 2
    wload(k0, tb0, sl0)
    wwrite(k0 - 2, rb0, sw0)
    _transpose_chunk(tb0, rb0)
    pltpu.sync_copy(rb0, dst(k0))
    wload(k0 + 1, tb1, sl1)
    wwrite(k0 - 1, rb1, sw1)
    _transpose_chunk(tb1, rb1)
    pltpu.sync_copy(rb1, dst(k0 + 1))

    @pl.when(wid < _N_FULL - _K1_MAIN * _NW)
    def _():
        j = _K1_MAIN * _NW + wid
        pltpu.sync_copy(tableT_hbm.at[:, pl.ds(j * 128, 128)], tb0)
        _transpose_chunk(tb0, rb0)
        pltpu.sync_copy(rb0, r_hbm.at[pl.ds(j * 64, 64), :])

    @pl.when(wid == 4)
    def _():
        pltpu.sync_copy(tail_hbm, rb1.at[pl.ds(0, _TAIL // 2), :])
        pltpu.sync_copy(rb1.at[pl.ds(0, _TAIL // 2), :],
                        r_hbm.at[pl.ds(_N_FULL * 64, _TAIL // 2), :])


@functools.partial(
    pl.kernel,
    mesh=_mesh,
    out_type=jax.ShapeDtypeStruct((_B, DIM), jnp.float32),
    scratch_types=[
        pltpu.VMEM((_B_PER_W,), jnp.int32),
        pltpu.VMEM((_CHUNK, DIM), jnp.float32),
        pltpu.VMEM((_CHUNK, DIM), jnp.float32),
        pltpu.SemaphoreType.DMA,
        pltpu.SemaphoreType.DMA,
    ],
    compiler_params=pltpu.CompilerParams(use_tc_tiling_on_sc=False),
)
def _gather_kernel(table_hbm, idx_hbm, out_hbm, idx_v, rows0, rows1, sem0, sem1):
    wid = lax.axis_index("s") * _NC + lax.axis_index("c")
    base = wid * _B_PER_W

    pltpu.sync_copy(idx_hbm.at[pl.ds(base, _B_PER_W)], idx_v)

    def start_gather(s, rows, sem):
        pltpu.async_copy(table_hbm.at[idx_v.at[pl.ds(s * _CHUNK, _CHUNK)]],
                         rows, sem)

    def finish(s, rows, sem):
        pltpu.make_async_copy(
            table_hbm.at[idx_v.at[pl.ds(s * _CHUNK, _CHUNK)]], rows, sem
        ).wait()
        pltpu.sync_copy(rows, out_hbm.at[pl.ds(base + s * _CHUNK, _CHUNK)])

    start_gather(0, rows0, sem0)

    def outer(o, _):
        s0 = 2 * o
        start_gather(s0 + 1, rows1, sem1)
        finish(s0, rows0, sem0)
        start_gather(s0 + 2, rows0, sem0)
        finish(s0 + 1, rows1, sem1)
        return _

    lax.fori_loop(0, _N_OUTER - 1, outer, None)

    s0 = _N_CHUNK - 2
    start_gather(s0 + 1, rows1, sem1)
    finish(s0, rows0, sem0)
    finish(s0 + 1, rows1, sem1)


_K3_BW = 2048
_K3_NB = BATCH // _K3_BW


def _eye64():
    return (lax.broadcasted_iota(jnp.int32, (DIM, DIM), 0)
            == lax.broadcasted_iota(jnp.int32, (DIM, DIM), 1)).astype(jnp.float32)


def _k3_body(g_ref, o_ref):
    eye = _eye64()
    dn = (((0,), (1,)), ((), ()))
    o_ref[0, :, 0:_K3_BW // 2] = lax.dot_general(
        eye, g_ref[:, 0:DIM], dn, preferred_element_type=jnp.float32)
    o_ref[0, :, _K3_BW // 2:_K3_BW] = lax.dot_general(
        eye, g_ref[:, DIM:128], dn, preferred_element_type=jnp.float32)


_k3_call = pl.pallas_call(
    _k3_body,
    grid=(HIST, _K3_NB),
    in_specs=[pl.BlockSpec((_K3_BW // 2, 128),
                           lambda h, jb: (h * _K3_NB + jb, 0))],
    out_specs=pl.BlockSpec((1, DIM, _K3_BW), lambda h, jb: (h, 0, jb)),
    out_shape=jax.ShapeDtypeStruct((HIST, DIM, BATCH), jnp.float32),
)


def kernel(x, embedding):
    tail = embedding[_N_FULL * 128:, :].reshape(_TAIL // 2, 128)
    r = _rebuild_kernel(embedding.T, tail)
    r2d = r.reshape(VOCAB, DIM)
    xperm = x.T.reshape(HIST, _K3_NB, 2, _K3_BW // 2)
    xperm = xperm.transpose(0, 1, 3, 2).reshape(_B)
    g = _gather_kernel(r2d, xperm)
    g2 = g.reshape(_B // 2, 128)
    outT = _k3_call(g2)
    return outT.transpose(2, 0, 1)


# speedup vs baseline: 1.5984x; 1.5874x over previous
import functools

import jax
import jax.numpy as jnp
from jax import lax
from jax.experimental import pallas as pl
from jax.experimental.pallas import tpu as pltpu
from jax.experimental.pallas import tpu_sc as plsc

VOCAB = 1000000
DIM = 64
BATCH = 16384
HIST = 50

_B = BATCH * HIST
_R_ROWS = VOCAB // 2

_info = plsc.get_sparse_core_info()
_NC, _NS = _info.num_cores, _info.num_subcores
_NW = _NC * _NS

_N_FULL = VOCAB // 128
_TAIL = VOCAB - _N_FULL * 128
_K1_MAIN = 244

_B_PER_W = _B // _NW
_CHUNK = 800
_N_CHUNK = _B_PER_W // _CHUNK
_N_OUTER = _N_CHUNK // 2

_mesh = plsc.VectorSubcoreMesh(core_axis_name="c", subcore_axis_name="s")


def _transpose_chunk(tbuf, rbuf):
    iota16 = lax.iota(jnp.int32, 16)

    def row_step(rr, _):
        for q in (0, 1):
            col = jnp.full((16,), 2 * rr + q, jnp.int32)
            for g in range(4):
                row = iota16 + (16 * g)
                v = plsc.load_gather(tbuf, [row, col])
                rbuf[rr, pl.ds(64 * q + 16 * g, 16)] = v
        return _

    lax.fori_loop(0, 64, row_step, None)


@functools.partial(
    pl.kernel,
    mesh=_mesh,
    out_type=jax.ShapeDtypeStruct((_R_ROWS, 128), jnp.float32),
    scratch_types=[
        pltpu.VMEM((64, 128), jnp.float32),
        pltpu.VMEM((64, 128), jnp.float32),
        pltpu.VMEM((64, 128), jnp.float32),
        pltpu.VMEM((64, 128), jnp.float32),
        pltpu.SemaphoreType.DMA,
        pltpu.SemaphoreType.DMA,
        pltpu.SemaphoreType.DMA,
        pltpu.SemaphoreType.DMA,
    ],
    compiler_params=pltpu.CompilerParams(use_tc_tiling_on_sc=True,
                                         needs_layout_passes=False),
)
def _rebuild_kernel(tableT_hbm, tail_hbm, r_hbm,
                    tb0, tb1, rb0, rb1, sl0, sl1, sw0, sw1):
    wid = lax.axis_index("s") * _NC + lax.axis_index("c")

    def src(k):
        return tableT_hbm.at[:, pl.ds((wid + _NW * k) * 128, 128)]

    def dst(k):
        return r_hbm.at[pl.ds((wid + _NW * k) * 64, 64), :]

    def load(k, tb, sl):
        pltpu.async_copy(src(k), tb, sl)

    def wload(k, tb, sl):
        pltpu.make_async_copy(src(k), tb, sl).wait()

    def write(k, rb, sw):
        pltpu.async_copy(rb, dst(k), sw)

    def wwrite(k, rb, sw):
        pltpu.make_async_copy(rb, dst(k), sw).wait()

    load(0, tb0, sl0)
    load(1, tb1, sl1)

    wload(0, tb0, sl0)
    _transpose_chunk(tb0, rb0)
    write(0, rb0, sw0)
    load(2, tb0, sl0)
    wload(1, tb1, sl1)
    _transpose_chunk(tb1, rb1)
    write(1, rb1, sw1)
    load(3, tb1, sl1)

    def outer(p, _):
        k0 = 2 * p
        wload(k0, tb0, sl0)
        wwrite(k0 - 2, rb0, sw0)
        _transpose_chunk(tb0, rb0)
        write(k0, rb0, sw0)
        load(k0 + 2, tb0, sl0)
        wload(k0 + 1, tb1, sl1)
        wwrite(k0 - 1, rb1, sw1)
        _transpose_chunk(tb1, rb1)
        write(k0 + 1, rb1, sw1)
        load(k0 + 3, tb1, sl1)
        return _

    lax.fori_loop(1, _K1_MAIN // 2 - 1, outer, None)

    k0 = _K1_MAIN - 2
    wload(k0, tb0, sl0)
    wwrite(k0 - 2, rb0, sw0)
    _transpose_chunk(tb0, rb0)
    pltpu.sync_copy(rb0, dst(k0))
    wload(k0 + 1, tb1, sl1)
    wwrite(k0 - 1, rb1, sw1)
    _transpose_chunk(tb1, rb1)
    pltpu.sync_copy(rb1, dst(k0 + 1))

    @pl.when(wid < _N_FULL - _K1_MAIN * _NW)
    def _():
        j = _K1_MAIN * _NW + wid
        pltpu.sync_copy(tableT_hbm.at[:, pl.ds(j * 128, 128)], tb0)
        _transpose_chunk(tb0, rb0)
        pltpu.sync_copy(rb0, r_hbm.at[pl.ds(j * 64, 64), :])

    @pl.when(wid == 4)
    def _():
        pltpu.sync_copy(tail_hbm, rb1.at[pl.ds(0, _TAIL // 2), :])
        pltpu.sync_copy(rb1.at[pl.ds(0, _TAIL // 2), :],
                        r_hbm.at[pl.ds(_N_FULL * 64, _TAIL // 2), :])


@functools.partial(
    pl.kernel,
    mesh=_mesh,
    out_type=jax.ShapeDtypeStruct((_B, DIM), jnp.float32),
    scratch_types=[
        pltpu.VMEM((_B_PER_W,), jnp.int32),
        pltpu.VMEM((_CHUNK, DIM), jnp.float32),
        pltpu.VMEM((_CHUNK, DIM), jnp.float32),
        pltpu.SemaphoreType.DMA,
        pltpu.SemaphoreType.DMA,
    ],
    compiler_params=pltpu.CompilerParams(use_tc_tiling_on_sc=False),
)
def _gather_kernel(table_hbm, idx_hbm, out_hbm, idx_v, rows0, rows1, sem0, sem1):
    wid = lax.axis_index("s") * _NC + lax.axis_index("c")
    base = wid * _B_PER_W

    pltpu.sync_copy(idx_hbm.at[pl.ds(base, _B_PER_W)], idx_v)

    def start_gather(s, rows, sem):
        pltpu.async_copy(table_hbm.at[idx_v.at[pl.ds(s * _CHUNK, _CHUNK)]],
                         rows, sem)

    def finish(s, rows, sem):
        pltpu.make_async_copy(
            table_hbm.at[idx_v.at[pl.ds(s * _CHUNK, _CHUNK)]], rows, sem
        ).wait()
        pltpu.sync_copy(rows, out_hbm.at[pl.ds(base + s * _CHUNK, _CHUNK)])

    start_gather(0, rows0, sem0)

    def outer(o, _):
        s0 = 2 * o
        start_gather(s0 + 1, rows1, sem1)
        finish(s0, rows0, sem0)
        start_gather(s0 + 2, rows0, sem0)
        finish(s0 + 1, rows1, sem1)
        return _

    lax.fori_loop(0, _N_OUTER - 1, outer, None)

    s0 = _N_CHUNK - 2
    start_gather(s0 + 1, rows1, sem1)
    finish(s0, rows0, sem0)
    finish(s0 + 1, rows1, sem1)


_K3_BW = 2048
_K3_NB = BATCH // _K3_BW


def _eye64():
    return (lax.broadcasted_iota(jnp.int32, (DIM, DIM), 0)
            == lax.broadcasted_iota(jnp.int32, (DIM, DIM), 1)).astype(jnp.float32)


def _k3_body(g_ref, o_ref):
    eye = _eye64()
    dn = (((0,), (1,)), ((), ()))
    o_ref[0, :, 0:_K3_BW // 2] = lax.dot_general(
        eye, g_ref[:, 0:DIM], dn, preferred_element_type=jnp.float32)
    o_ref[0, :, _K3_BW // 2:_K3_BW] = lax.dot_general(
        eye, g_ref[:, DIM:128], dn, preferred_element_type=jnp.float32)


_k3_call = pl.pallas_call(
    _k3_body,
    grid=(HIST, _K3_NB),
    in_specs=[pl.BlockSpec((_K3_BW // 2, 128),
                           lambda h, jb: (h * _K3_NB + jb, 0))],
    out_specs=pl.BlockSpec((1, DIM, _K3_BW), lambda h, jb: (h, 0, jb)),
    out_shape=jax.ShapeDtypeStruct((HIST, DIM, BATCH), jnp.float32),
)


def kernel(x, embedding):
    xperm = x.T.reshape(HIST, _K3_NB, 2, _K3_BW // 2)
    xperm = xperm.transpose(0, 1, 3, 2).reshape(_B)
    g = _gather_kernel(embedding, xperm)
    g2 = g.reshape(_B // 2, 128)
    outT = _k3_call(g2)
    return outT.transpose(2, 0, 1)
